# Initial kernel scaffold; baseline (speedup 1.0000x reference)
import jax
import jax.numpy as jnp
from jax.experimental import pallas as pl


def kernel(x, edge_attr, receivers, Wn, bn, We, be, Wm0, bm0, Wm1, bm1, Wu0, bu0, Wu1, bu1, Wc1, bc1, Wc2, bc2):
    raise NotImplementedError("write your pallas kernel here")


# speedup vs baseline: 11.4018x; 11.4018x over previous
import functools

import jax
import jax.numpy as jnp
from jax import lax
from jax.experimental import pallas as pl
from jax.experimental.pallas import tpu as pltpu
from jax.experimental.pallas import tpu_sc as plsc

NC = 2
NS = 16
L = 16
NW = NC * NS
SUPER = 25
CAP = 16


def _sc_scan_body(recv_hbm, attr_hbm, rows_out, cnt_out, ovf_out,
                  recv_v, buf_v, acc_v, cnt_v, row_v, slot_ref):
    E = recv_hbm.shape[0]
    chunk = E // NW
    nsuper = chunk // (L * SUPER)
    wid = lax.axis_index("s") * NC + lax.axis_index("c")
    base = wid * chunk

    pltpu.sync_copy(recv_hbm.at[pl.ds(base, chunk)], recv_v)
    for j in range(CAP):
        buf_v[j, :] = jnp.zeros((L,), jnp.float32)
    acc_v[...] = jnp.zeros((L,), jnp.float32)
    cnt_v[...] = jnp.zeros((L,), jnp.int32)
    slot_ref[0] = 0

    def scan_group(off):
        vvec = recv_v[pl.ds(off, L)]
        for l in range(L):
            @pl.when(vvec[l] == 0)
            def _grab_row():
                slot = slot_ref[0]

                @pl.when(slot < CAP)
                def _buffer():
                    pltpu.sync_copy(attr_hbm.at[base + off + l],
                                    buf_v.at[slot])

                @pl.when(slot >= CAP)
                def _overflow():
                    pltpu.sync_copy(attr_hbm.at[base + off + l], row_v)
                    acc_v[...] = acc_v[...] + row_v[...]

                slot_ref[0] = slot + 1
                cnt_v[...] = cnt_v[...] + 1

    def super_body(sg, carry):
        off0 = sg * (L * SUPER)
        minv = recv_v[pl.ds(off0, L)]
        for u in range(1, SUPER):
            minv = jnp.minimum(minv, recv_v[pl.ds(off0 + u * L, L)])
        m = minv[0]
        for l in range(1, L):
            m = jnp.minimum(m, minv[l])

        @pl.when(m == 0)
        def _slow():
            def inner(u, c2):
                scan_group(off0 + u * L)
                return c2
            lax.fori_loop(0, SUPER, inner, 0)
        return carry

    lax.fori_loop(0, nsuper, super_body, 0)
    pltpu.sync_copy(buf_v, rows_out.at[wid])
    pltpu.sync_copy(cnt_v, cnt_out.at[wid])
    pltpu.sync_copy(acc_v, ovf_out.at[wid])


def _bfr(a):
    return a.astype(jnp.bfloat16).astype(jnp.float32)


def _head_kernel(rows_ref, cntf_ref, ovf_ref, x0_ref, Wn_ref, bn_ref,
                 We_ref, be_ref, Wm0_ref, bm0_ref, Wm1_ref, bm1_ref,
                 Wu0_ref, bu0_ref, Wu1_ref, bu1_ref,
                 Wc1_ref, Wc1b_ref, Wc1c_ref, bc1_ref,
                 Wc2_ref, Wc2b_ref, Wc2c_ref, bc2_ref, out_ref):
    dot = functools.partial(jax.lax.dot_general,
                            dimension_numbers=(((1,), (0,)), ((), ())),
                            preferred_element_type=jnp.float32)

    def split3(a):
        hi = _bfr(a)
        lo = _bfr(a - hi)
        l2 = _bfr(a - hi - lo)
        return hi, lo, l2

    def dot_x(a, w):
        hi, lo, l2 = split3(a)
        return dot(hi, w) + dot(lo, w) + dot(l2, w)

    def dot_xx(a, wparts):
        aparts = split3(a)
        out = None
        for ap in aparts:
            for wp in wparts:
                t = dot(ap, wp)
                out = t if out is None else out + t
        return out

    cntf = cntf_ref[...]
    rows = rows_ref[...]
    slot = jax.lax.broadcasted_iota(jnp.int32, (NW * CAP, 1), 0) % CAP
    valid = slot.astype(jnp.float32) < cntf
    c = (jnp.sum(cntf) / CAP).reshape(1, 1)

    edges_rows = dot(_bfr(rows), We_ref[...]) + be_ref[...]
    edges_rb = _bfr(edges_rows)
    S2 = jnp.sum(jnp.where(valid, edges_rb, 0.0), axis=0, keepdims=True)

    s_ovf = jnp.sum(ovf_ref[...], axis=0, keepdims=True)
    c_ovf = (jnp.sum(jnp.maximum(cntf - CAP, 0.0)) / CAP).reshape(1, 1)
    S2 = S2 + _bfr(dot_x(s_ovf, We_ref[...]) + c_ovf * be_ref[...])

    agg1 = dot_x(S2, Wm0_ref[...]) + c * bm0_ref[...]
    agg2 = dot_x(S2, Wm1_ref[...]) + c * bm1_ref[...]
    n0 = dot(_bfr(x0_ref[...]), Wn_ref[...]) + bn_ref[...]
    n1 = jnp.maximum(dot(_bfr(n0 + agg1), Wu0_ref[...]) + bu0_ref[...], 0.0)
    n2 = jnp.maximum(dot(_bfr(n1 + agg2), Wu1_ref[...]) + bu1_ref[...], 0.0)
    h = jnp.maximum(dot_xx(n2, (Wc1_ref[...], Wc1b_ref[...], Wc1c_ref[...]))
                    + bc1_ref[...], 0.0)
    out_ref[...] = (dot_xx(h, (Wc2_ref[...], Wc2b_ref[...], Wc2c_ref[...]))
                    + bc2_ref[...])


def kernel(x, edge_attr, receivers, Wn, bn, We, be, Wm0, bm0, Wm1, bm1,
           Wu0, bu0, Wu1, bu1, Wc1, bc1, Wc2, bc2):
    mesh = plsc.VectorSubcoreMesh(core_axis_name="c", subcore_axis_name="s",
                                  num_cores=NC, num_subcores=NS)
    E = receivers.shape[0]
    rows, cnt, ovf = pl.kernel(
        _sc_scan_body,
        out_type=[jax.ShapeDtypeStruct((NW, CAP, L), jnp.float32),
                  jax.ShapeDtypeStruct((NW, L), jnp.int32),
                  jax.ShapeDtypeStruct((NW, L), jnp.float32)],
        mesh=mesh,
        scratch_types=[
            pltpu.VMEM((E // NW,), jnp.int32),
            pltpu.VMEM((CAP, L), jnp.float32),
            pltpu.VMEM((L,), jnp.float32),
            pltpu.VMEM((L,), jnp.int32),
            pltpu.VMEM((L,), jnp.float32),
            pltpu.SMEM((1,), jnp.int32),
        ],
    )(receivers, edge_attr)

    rows2d = rows.reshape(NW * CAP, L)
    cntf = cnt.reshape(NW * CAP, 1).astype(jnp.float32)

    bfr = lambda a: a.astype(jnp.bfloat16).astype(jnp.float32)

    def wsplit3(w):
        hi = bfr(w)
        lo = bfr(w - hi)
        l2 = bfr(w - hi - lo)
        return hi, lo, l2

    Wc1a, Wc1b, Wc1c = wsplit3(Wc1)
    Wc2a, Wc2b, Wc2c = wsplit3(Wc2)
    out = pl.pallas_call(
        _head_kernel,
        out_shape=jax.ShapeDtypeStruct((1, 1), jnp.float32),
    )(rows2d, cntf, ovf, x[0:1], bfr(Wn), bn.reshape(1, 64),
      bfr(We), be.reshape(1, 64),
      bfr(Wm0), bm0.reshape(1, 64), bfr(Wm1), bm1.reshape(1, 64),
      bfr(Wu0), bu0.reshape(1, 64), bfr(Wu1), bu1.reshape(1, 64),
      Wc1a, Wc1b, Wc1c, bc1.reshape(1, 32),
      Wc2a, Wc2b, Wc2c, bc2.reshape(1, 1))
    return out[0, 0]


# speedup vs baseline: 14.6599x; 1.2858x over previous
import functools

import jax
import jax.numpy as jnp
from jax import lax
from jax.experimental import pallas as pl
from jax.experimental.pallas import tpu as pltpu
from jax.experimental.pallas import tpu_sc as plsc

NC = 2
NS = 16
L = 16
NW = NC * NS
SUPER = 25
CAP = 16


def _sc_scan_body(recv_hbm, attr_hbm, rows_out, cnt_out, ovf_out,
                  recv_v, buf_v, acc_v, cnt_v, row_v, rot_v, slot_ref):
    E = recv_hbm.shape[0]
    chunk = E // NW
    nsuper = chunk // (L * SUPER)
    wid = lax.axis_index("s") * NC + lax.axis_index("c")
    base = wid * chunk

    pltpu.sync_copy(recv_hbm.at[pl.ds(base, chunk)], recv_v)
    for j in range(CAP):
        buf_v[j, :] = jnp.zeros((L,), jnp.float32)
    acc_v[...] = jnp.zeros((L,), jnp.float32)
    cnt_v[...] = jnp.zeros((L,), jnp.int32)
    slot_ref[0] = 0

    def lane_min(m):
        for k in (8, 4, 2, 1):
            rot_v[pl.ds(0, L)] = m
            rot_v[pl.ds(L, L)] = m
            m = jnp.minimum(m, rot_v[pl.ds(L - k, L)])
        return m[0]

    def scan_group(off, vvec):
        for l in range(L):
            @pl.when(vvec[l] == 0)
            def _grab_row():
                slot = slot_ref[0]

                @pl.when(slot < CAP)
                def _buffer():
                    pltpu.sync_copy(attr_hbm.at[base + off + l],
                                    buf_v.at[slot])

                @pl.when(slot >= CAP)
                def _overflow():
                    pltpu.sync_copy(attr_hbm.at[base + off + l], row_v)
                    acc_v[...] = acc_v[...] + row_v[...]

                slot_ref[0] = slot + 1
                cnt_v[...] = cnt_v[...] + 1

    def super_body(sg, carry):
        off0 = sg * (L * SUPER)
        vs = [recv_v[pl.ds(off0 + u * L, L)] for u in range(SUPER)]
        while len(vs) > 1:
            folded = [jnp.minimum(a, b) for a, b in zip(vs[::2], vs[1::2])]
            if len(vs) % 2:
                folded.append(vs[-1])
            vs = folded

        @pl.when(lane_min(vs[0]) == 0)
        def _slow():
            def inner(u, c2):
                off = off0 + u * L
                g = recv_v[pl.ds(off, L)]

                @pl.when(lane_min(g) == 0)
                def _scan():
                    scan_group(off, g)
                return c2
            lax.fori_loop(0, SUPER, inner, 0)
        return carry

    lax.fori_loop(0, nsuper, super_body, 0)
    pltpu.sync_copy(buf_v, rows_out.at[wid])
    pltpu.sync_copy(cnt_v, cnt_out.at[wid])
    pltpu.sync_copy(acc_v, ovf_out.at[wid])


def _bfr(a):
    return a.astype(jnp.bfloat16).astype(jnp.float32)


def _head_kernel(rows_ref, cnt_ref, ovf_ref, x0_ref, Wn_ref, bn_ref,
                 We_ref, be_ref, Wm0_ref, bm0_ref, Wm1_ref, bm1_ref,
                 Wu0_ref, bu0_ref, Wu1_ref, bu1_ref, Wc1_ref, bc1_ref,
                 Wc2_ref, bc2_ref, out_ref):
    dot = functools.partial(jax.lax.dot_general,
                            dimension_numbers=(((1,), (0,)), ((), ())),
                            preferred_element_type=jnp.float32)

    def split3(a):
        hi = _bfr(a)
        lo = _bfr(a - hi)
        l2 = _bfr(a - hi - lo)
        return hi, lo, l2

    def dot_x(a, w):
        hi, lo, l2 = split3(a)
        return dot(hi, w) + dot(lo, w) + dot(l2, w)

    def dot_xx(a, wparts):
        aparts = split3(a)
        out = None
        for ap in aparts:
            for wp in wparts:
                t = dot(ap, wp)
                out = t if out is None else out + t
        return out

    cntf = cnt_ref[...].astype(jnp.float32)
    rows = rows_ref[...]
    slot = jax.lax.broadcasted_iota(jnp.int32, (NW * CAP, 1), 0) % CAP
    valid = slot.astype(jnp.float32) < cntf
    c = (jnp.sum(cntf) / CAP).reshape(1, 1)

    We_r = _bfr(We_ref[...])
    be = be_ref[...]

    edges_rows = dot(_bfr(rows), We_r) + be
    edges_rb = _bfr(edges_rows)
    S2 = jnp.sum(jnp.where(valid, edges_rb, 0.0), axis=0, keepdims=True)

    s_ovf = jnp.sum(ovf_ref[...], axis=0, keepdims=True)
    c_ovf = (jnp.sum(jnp.maximum(cntf - CAP, 0.0)) / CAP).reshape(1, 1)
    S2 = S2 + _bfr(dot_x(s_ovf, We_r) + c_ovf * be)

    agg1 = dot_x(S2, _bfr(Wm0_ref[...])) + c * bm0_ref[...]
    agg2 = dot_x(S2, _bfr(Wm1_ref[...])) + c * bm1_ref[...]
    n0 = dot(_bfr(x0_ref[...]), _bfr(Wn_ref[...])) + bn_ref[...]
    n1 = jnp.maximum(dot(_bfr(n0 + agg1), _bfr(Wu0_ref[...])) + bu0_ref[...],
                     0.0)
    n2 = jnp.maximum(dot(_bfr(n1 + agg2), _bfr(Wu1_ref[...])) + bu1_ref[...],
                     0.0)
    h = jnp.maximum(dot_xx(n2, split3(Wc1_ref[...])) + bc1_ref[...], 0.0)
    out_ref[...] = dot_xx(h, split3(Wc2_ref[...])) + bc2_ref[...]


def kernel(x, edge_attr, receivers, Wn, bn, We, be, Wm0, bm0, Wm1, bm1,
           Wu0, bu0, Wu1, bu1, Wc1, bc1, Wc2, bc2):
    mesh = plsc.VectorSubcoreMesh(core_axis_name="c", subcore_axis_name="s",
                                  num_cores=NC, num_subcores=NS)
    E = receivers.shape[0]
    rows, cnt, ovf = pl.kernel(
        _sc_scan_body,
        out_type=[jax.ShapeDtypeStruct((NW, CAP, L), jnp.float32),
                  jax.ShapeDtypeStruct((NW, L), jnp.int32),
                  jax.ShapeDtypeStruct((NW, L), jnp.float32)],
        mesh=mesh,
        scratch_types=[
            pltpu.VMEM((E // NW,), jnp.int32),
            pltpu.VMEM((CAP, L), jnp.float32),
            pltpu.VMEM((L,), jnp.float32),
            pltpu.VMEM((L,), jnp.int32),
            pltpu.VMEM((L,), jnp.float32),
            pltpu.VMEM((2 * L,), jnp.int32),
            pltpu.SMEM((1,), jnp.int32),
        ],
    )(receivers, edge_attr)

    rows2d = rows.reshape(NW * CAP, L)
    cnt2d = cnt.reshape(NW * CAP, 1)

    out = pl.pallas_call(
        _head_kernel,
        out_shape=jax.ShapeDtypeStruct((1, 1), jnp.float32),
    )(rows2d, cnt2d, ovf, x[0:1], Wn, bn.reshape(1, 64),
      We, be.reshape(1, 64),
      Wm0, bm0.reshape(1, 64), Wm1, bm1.reshape(1, 64),
      Wu0, bu0.reshape(1, 64), Wu1, bu1.reshape(1, 64),
      Wc1, bc1.reshape(1, 32), Wc2, bc2.reshape(1, 1))
    return out[0, 0]


# speedup vs baseline: 14.7055x; 1.0031x over previous
import functools

import jax
import jax.numpy as jnp
from jax import lax
from jax.experimental import pallas as pl
from jax.experimental.pallas import tpu as pltpu
from jax.experimental.pallas import tpu_sc as plsc

NC = 2
NS = 16
L = 16
NW = NC * NS
SUPER = 25
CAP = 16


def _sc_scan_body(recv_hbm, attr_hbm, rows_out, cnt_out, ovf_out,
                  recv_v, buf_v, acc_v, cnt_v, row_v, rot_v, slot_ref):
    E = recv_hbm.shape[0]
    chunk = E // NW
    nsuper = chunk // (L * SUPER)
    wid = lax.axis_index("s") * NC + lax.axis_index("c")
    base = wid * chunk

    pltpu.sync_copy(recv_hbm.at[pl.ds(base, chunk)], recv_v)
    for j in range(CAP):
        buf_v[j, :] = jnp.zeros((L,), jnp.float32)
    acc_v[...] = jnp.zeros((L,), jnp.float32)
    cnt_v[...] = jnp.zeros((L,), jnp.int32)
    slot_ref[0] = 0

    def lane_min(m):
        for k in (8, 4, 2, 1):
            rot_v[pl.ds(0, L)] = m
            rot_v[pl.ds(L, L)] = m
            m = jnp.minimum(m, rot_v[pl.ds(L - k, L)])
        return m[0]

    def scan_group(off, vvec):
        for l in range(L):
            @pl.when(vvec[l] == 0)
            def _grab_row():
                slot = slot_ref[0]

                @pl.when(slot < CAP)
                def _buffer():
                    pltpu.sync_copy(attr_hbm.at[base + off + l],
                                    buf_v.at[slot])

                @pl.when(slot >= CAP)
                def _overflow():
                    pltpu.sync_copy(attr_hbm.at[base + off + l], row_v)
                    acc_v[...] = acc_v[...] + row_v[...]

                slot_ref[0] = slot + 1
                cnt_v[...] = cnt_v[...] + 1

    def super_body(sg, carry):
        off0 = sg * (L * SUPER)
        vs = [recv_v[pl.ds(off0 + u * L, L)] for u in range(SUPER)]
        while len(vs) > 1:
            folded = [jnp.minimum(a, b) for a, b in zip(vs[::2], vs[1::2])]
            if len(vs) % 2:
                folded.append(vs[-1])
            vs = folded

        @pl.when(lane_min(vs[0]) == 0)
        def _slow():
            def inner(u, c2):
                off = off0 + u * L
                g = recv_v[pl.ds(off, L)]

                @pl.when(lane_min(g) == 0)
                def _scan():
                    scan_group(off, g)
                return c2
            lax.fori_loop(0, SUPER, inner, 0)
        return carry

    lax.fori_loop(0, nsuper, super_body, 0)
    pltpu.sync_copy(buf_v, rows_out.at[wid])
    pltpu.sync_copy(cnt_v, cnt_out.at[wid])
    pltpu.sync_copy(acc_v, ovf_out.at[wid])


def _bfr(a):
    return a.astype(jnp.bfloat16).astype(jnp.float32)


def _head_kernel(rows_ref, cnt_ref, ovf_ref, x0_ref, Wn_ref, bn_ref,
                 We_ref, be_ref, Wm0_ref, bm0_ref, Wm1_ref, bm1_ref,
                 Wu0_ref, bu0_ref, Wu1_ref, bu1_ref, Wc1_ref, bc1_ref,
                 Wc2_ref, bc2_ref, out_ref):
    dot = functools.partial(jax.lax.dot_general,
                            dimension_numbers=(((1,), (0,)), ((), ())),
                            preferred_element_type=jnp.float32)

    def split3(a):
        hi = _bfr(a)
        lo = _bfr(a - hi)
        l2 = _bfr(a - hi - lo)
        return hi, lo, l2

    def dot_x(a, w):
        hi, lo, l2 = split3(a)
        return dot(hi, w) + dot(lo, w) + dot(l2, w)

    def dot_xx(a, wparts):
        aparts = split3(a)
        out = None
        for ap in aparts:
            for wp in wparts:
                t = dot(ap, wp)
                out = t if out is None else out + t
        return out

    cntf = cnt_ref[...].astype(jnp.float32)
    rows = rows_ref[...]
    slot = jax.lax.broadcasted_iota(jnp.int32, (NW * CAP, 1), 0) % CAP
    valid = slot.astype(jnp.float32) < cntf
    c = (jnp.sum(cntf) / CAP).reshape(1, 1)

    We_r = _bfr(We_ref[...])
    be = be_ref[...]

    edges_rows = dot(_bfr(rows), We_r) + be
    edges_rb = _bfr(edges_rows)
    S2 = jnp.sum(jnp.where(valid, edges_rb, 0.0), axis=0, keepdims=True)

    s_ovf = jnp.sum(ovf_ref[...], axis=0, keepdims=True)
    c_ovf = (jnp.sum(jnp.maximum(cntf - CAP, 0.0)) / CAP).reshape(1, 1)
    S2 = S2 + _bfr(dot_x(s_ovf, We_r) + c_ovf * be)

    agg1 = dot_x(S2, _bfr(Wm0_ref[...])) + c * bm0_ref[...]
    agg2 = dot_x(S2, _bfr(Wm1_ref[...])) + c * bm1_ref[...]
    n0 = dot(_bfr(x0_ref[...]), _bfr(Wn_ref[...])) + bn_ref[...]
    n1 = jnp.maximum(dot(_bfr(n0 + agg1), _bfr(Wu0_ref[...])) + bu0_ref[...],
                     0.0)
    n2 = jnp.maximum(dot(_bfr(n1 + agg2), _bfr(Wu1_ref[...])) + bu1_ref[...],
                     0.0)
    h = jnp.maximum(dot_xx(n2, split3(Wc1_ref[...])) + bc1_ref[...], 0.0)
    out_ref[...] = dot_xx(h, split3(Wc2_ref[...])) + bc2_ref[...]


def kernel(x, edge_attr, receivers, Wn, bn, We, be, Wm0, bm0, Wm1, bm1,
           Wu0, bu0, Wu1, bu1, Wc1, bc1, Wc2, bc2):
    mesh = plsc.VectorSubcoreMesh(core_axis_name="c", subcore_axis_name="s",
                                  num_cores=NC, num_subcores=NS)
    E = receivers.shape[0]
    rows, cnt, ovf = pl.kernel(
        _sc_scan_body,
        out_type=[jax.ShapeDtypeStruct((NW, CAP, L), jnp.float32),
                  jax.ShapeDtypeStruct((NW, L), jnp.int32),
                  jax.ShapeDtypeStruct((NW, L), jnp.float32)],
        mesh=mesh,
        compiler_params=pltpu.CompilerParams(use_tc_tiling_on_sc=True),
        scratch_types=[
            pltpu.VMEM((E // NW,), jnp.int32),
            pltpu.VMEM((CAP, L), jnp.float32),
            pltpu.VMEM((L,), jnp.float32),
            pltpu.VMEM((L,), jnp.int32),
            pltpu.VMEM((L,), jnp.float32),
            pltpu.VMEM((2 * L,), jnp.int32),
            pltpu.SMEM((1,), jnp.int32),
        ],
    )(receivers, edge_attr)

    rows2d = rows.reshape(NW * CAP, L)
    cnt2d = cnt.reshape(NW * CAP, 1)

    out = pl.pallas_call(
        _head_kernel,
        out_shape=jax.ShapeDtypeStruct((1, 1), jnp.float32),
    )(rows2d, cnt2d, ovf, x[0:1], Wn, bn.reshape(1, 64),
      We, be.reshape(1, 64),
      Wm0, bm0.reshape(1, 64), Wm1, bm1.reshape(1, 64),
      Wu0, bu0.reshape(1, 64), Wu1, bu1.reshape(1, 64),
      Wc1, bc1.reshape(1, 32), Wc2, bc2.reshape(1, 1))
    return out[0, 0]
